# Initial kernel scaffold; baseline (speedup 1.0000x reference)
import jax
import jax.numpy as jnp
from jax.experimental import pallas as pl


def kernel(embedding, emb_table, W1, b1, W2, b2, Wout, bout):
    raise NotImplementedError("write your pallas kernel here")


# speedup vs baseline: 13.3100x; 13.3100x over previous
import jax
import jax.numpy as jnp
from jax.experimental import pallas as pl
from jax.experimental.pallas import tpu as pltpu

D_DEPTH = 16
L_WIDTH = 1024
P_PAR = 16
H = 64
E = 64
BATCH = 32
TL = 256
NT = L_WIDTH // TL


def _dag_body(emb_ref, table_ref, w1p_ref, w1e_ref, b1_ref, w2_ref, b2_ref,
              woutm_ref, boutm_ref, woutr_ref, boutr_ref,
              outm_ref, outr_ref, slab_ref):
    d = pl.program_id(0)
    t = pl.program_id(1)

    @pl.when(jnp.logical_and(d == 0, t == 0))
    def _root_out():
        outr_ref[:] = (jnp.sum(emb_ref[:] * woutr_ref[:], axis=1,
                               keepdims=True) + boutr_ref[:])

    r = (d + 1) % 2
    w = d % 2

    base = t * TL
    nxt = ((t + 1) % NT) * TL
    x = slab_ref[r, :, pl.ds(base, TL), :]
    x16 = slab_ref[r, :, pl.ds(nxt, 16), :]
    s = jnp.concatenate([x, x16], axis=1)
    for k in (1, 2, 4, 8):
        s = s[:, :-k, :] + s[:, k:, :]
    pv_win = s[:, :TL, :]

    pv = jnp.where(d == 0,
                   jnp.broadcast_to(emb_ref[:][:, None, :], (BATCH, TL, H)),
                   pv_win)

    np_tile = jax.lax.dot_general(
        table_ref[:], w1e_ref[:],
        dimension_numbers=(((1,), (1,)), ((), ())),
        preferred_element_type=jnp.float32) + b1_ref[:]
    pv2 = pv.reshape(BATCH * TL, H)
    h1 = jax.lax.dot_general(
        pv2, w1p_ref[:],
        dimension_numbers=(((1,), (1,)), ((), ())),
        preferred_element_type=jnp.float32).reshape(BATCH, TL, H)
    h1 = h1 + np_tile[None]
    a = jnp.maximum(h1, 0.0).reshape(BATCH * TL, H)
    h2 = jax.lax.dot_general(
        a, w2_ref[:],
        dimension_numbers=(((1,), (1,)), ((), ())),
        preferred_element_type=jnp.float32).reshape(BATCH, TL, H)
    cur = pv + h2 + b2_ref[:][None]

    slab_ref[w, :, pl.ds(base, TL), :] = cur
    outm_ref[0] = (jnp.sum(cur * woutm_ref[:][None], axis=2)
                   + boutm_ref[0, 0][None, :])


def _run(emb, table, w1p, w1e, b1, w2, b2, woutm, boutm, woutr, boutr):
    grid = (D_DEPTH, NT)
    outm, outr = pl.pallas_call(
        _dag_body,
        grid=grid,
        in_specs=[
            pl.BlockSpec((BATCH, H), lambda d, t: (0, 0)),
            pl.BlockSpec((TL, E), lambda d, t: (d * NT + t, 0)),
            pl.BlockSpec((H, H), lambda d, t: (0, 0)),
            pl.BlockSpec((H, E), lambda d, t: (0, 0)),
            pl.BlockSpec((1, H), lambda d, t: (0, 0)),
            pl.BlockSpec((H, H), lambda d, t: (0, 0)),
            pl.BlockSpec((1, H), lambda d, t: (0, 0)),
            pl.BlockSpec((TL, H), lambda d, t: (d * NT + t, 0)),
            pl.BlockSpec((1, 1, TL), lambda d, t: (d * NT + t, 0, 0)),
            pl.BlockSpec((1, H), lambda d, t: (0, 0)),
            pl.BlockSpec((1, 1), lambda d, t: (0, 0)),
        ],
        out_specs=[
            pl.BlockSpec((1, BATCH, TL), lambda d, t: (d, 0, t)),
            pl.BlockSpec((BATCH, 1), lambda d, t: (0, 0)),
        ],
        out_shape=[
            jax.ShapeDtypeStruct((D_DEPTH, BATCH, L_WIDTH), jnp.float32),
            jax.ShapeDtypeStruct((BATCH, 1), jnp.float32),
        ],
        scratch_shapes=[pltpu.VMEM((2, BATCH, L_WIDTH, H), jnp.float32)],
    )(emb, table, w1p, w1e, b1, w2, b2, woutm, boutm, woutr, boutr)
    return outm, outr


def kernel(embedding, emb_table, W1, b1, W2, b2, Wout, bout):
    table = emb_table[2:2 + D_DEPTH * L_WIDTH]
    w1p = W1[:, :H]
    w1e = W1[:, H:]
    woutm = Wout[0, 1:, :]
    boutm = bout[0, 1:].reshape(D_DEPTH * NT, 1, TL)
    woutr = Wout[0, 0:1, :]
    boutr = bout[:, 0:1]
    outm, outr = _run(embedding, table, w1p, w1e, b1.reshape(1, H), W2,
                      b2.reshape(1, H), woutm, boutm, woutr, boutr)
    out_main = outm.transpose(1, 0, 2).reshape(BATCH, D_DEPTH * L_WIDTH)
    return jnp.concatenate([outr, out_main], axis=1)


# speedup vs baseline: 17.4496x; 1.3110x over previous
import jax
import jax.numpy as jnp
from jax.experimental import pallas as pl
from jax.experimental.pallas import tpu as pltpu

D_DEPTH = 16
L_WIDTH = 1024
P_PAR = 16
H = 64
E = 64
BATCH = 32
TL = 512
NT = L_WIDTH // TL


def _dag_body(emb_ref, table_ref, w1p_ref, w1e_ref, b1_ref, w2_ref, b2_ref,
              woutm_ref, boutm_ref, woutr_ref, boutr_ref,
              outm_ref, outr_ref, slab_ref):
    d = pl.program_id(0)
    t = pl.program_id(1)

    @pl.when(jnp.logical_and(d == 0, t == 0))
    def _root_out():
        outr_ref[:] = (jnp.sum(emb_ref[:] * woutr_ref[:], axis=1,
                               keepdims=True) + boutr_ref[:])

    r = (d + 1) % 2
    w = d % 2

    base = t * TL
    nxt = ((t + 1) % NT) * TL

    @pl.when(d == 0)
    def _seed():
        seed = jnp.broadcast_to((emb_ref[:] * 0.0625)[:, None, :],
                                (BATCH, TL, H))
        slab_ref[r, :, pl.ds(base, TL), :] = seed
        slab_ref[r, :, pl.ds(nxt, 16), :] = seed[:, :16, :]

    x = slab_ref[r, :, pl.ds(base, TL), :]
    x16 = slab_ref[r, :, pl.ds(nxt, 16), :]
    s = jnp.concatenate([x, x16], axis=1)
    for k in (1, 2, 4, 8):
        s = s[:, :-k, :] + s[:, k:, :]
    pv = s[:, :TL, :]

    np_tile = jax.lax.dot_general(
        table_ref[:], w1e_ref[:],
        dimension_numbers=(((1,), (1,)), ((), ())),
        preferred_element_type=jnp.float32) + b1_ref[:]
    pv2 = pv.reshape(BATCH * TL, H)
    h1 = jax.lax.dot_general(
        pv2, w1p_ref[:],
        dimension_numbers=(((1,), (1,)), ((), ())),
        preferred_element_type=jnp.float32).reshape(BATCH, TL, H)
    h1 = h1 + np_tile[None]
    a = jnp.maximum(h1, 0.0).reshape(BATCH * TL, H)
    h2 = jax.lax.dot_general(
        a, w2_ref[:],
        dimension_numbers=(((1,), (1,)), ((), ())),
        preferred_element_type=jnp.float32).reshape(BATCH, TL, H)
    cur = pv + h2 + b2_ref[:][None]

    slab_ref[w, :, pl.ds(base, TL), :] = cur
    outm_ref[0] = (jnp.sum(cur * woutm_ref[:][None], axis=2)
                   + boutm_ref[0, 0][None, :])


def _run(emb, table, w1p, w1e, b1, w2, b2, woutm, boutm, woutr, boutr):
    grid = (D_DEPTH, NT)
    outm, outr = pl.pallas_call(
        _dag_body,
        grid=grid,
        in_specs=[
            pl.BlockSpec((BATCH, H), lambda d, t: (0, 0)),
            pl.BlockSpec((TL, E), lambda d, t: (d * NT + t, 0)),
            pl.BlockSpec((H, H), lambda d, t: (0, 0)),
            pl.BlockSpec((H, E), lambda d, t: (0, 0)),
            pl.BlockSpec((1, H), lambda d, t: (0, 0)),
            pl.BlockSpec((H, H), lambda d, t: (0, 0)),
            pl.BlockSpec((1, H), lambda d, t: (0, 0)),
            pl.BlockSpec((TL, H), lambda d, t: (d * NT + t, 0)),
            pl.BlockSpec((1, 1, TL), lambda d, t: (d * NT + t, 0, 0)),
            pl.BlockSpec((1, H), lambda d, t: (0, 0)),
            pl.BlockSpec((1, 1), lambda d, t: (0, 0)),
        ],
        out_specs=[
            pl.BlockSpec((1, BATCH, TL), lambda d, t: (d, 0, t)),
            pl.BlockSpec((BATCH, 1), lambda d, t: (0, 0)),
        ],
        out_shape=[
            jax.ShapeDtypeStruct((D_DEPTH, BATCH, L_WIDTH), jnp.float32),
            jax.ShapeDtypeStruct((BATCH, 1), jnp.float32),
        ],
        scratch_shapes=[pltpu.VMEM((2, BATCH, L_WIDTH, H), jnp.float32)],
    )(emb, table, w1p, w1e, b1, w2, b2, woutm, boutm, woutr, boutr)
    return outm, outr


def kernel(embedding, emb_table, W1, b1, W2, b2, Wout, bout):
    table = emb_table[2:2 + D_DEPTH * L_WIDTH]
    w1p = W1[:, :H]
    w1e = W1[:, H:]
    woutm = Wout[0, 1:, :]
    boutm = bout[0, 1:].reshape(D_DEPTH * NT, 1, TL)
    woutr = Wout[0, 0:1, :]
    boutr = bout[:, 0:1]
    outm, outr = _run(embedding, table, w1p, w1e, b1.reshape(1, H), W2,
                      b2.reshape(1, H), woutm, boutm, woutr, boutr)
    out_main = outm.transpose(1, 0, 2).reshape(BATCH, D_DEPTH * L_WIDTH)
    return jnp.concatenate([outr, out_main], axis=1)
